# Initial kernel scaffold; baseline (speedup 1.0000x reference)
import jax
import jax.numpy as jnp
from jax.experimental import pallas as pl


def kernel(x, edge_index, W_in, b_in, W_c0, b_c0, W_c1, b_c1, g0, be0, g1, be1, W_out, b_out):
    raise NotImplementedError("write your pallas kernel here")


# speedup vs baseline: 4.3738x; 4.3738x over previous
import functools

import jax
import jax.numpy as jnp
from jax import lax
from jax.experimental import pallas as pl
from jax.experimental.pallas import tpu as pltpu
from jax.experimental.pallas import tpu_sc as plsc

N = 10000
E = 320000
D = 128

NC = 2
NS = 16
NW = NC * NS
EPW = E // NW
C = 80
ITERS = EPW // C
N_PAD = 10240
RPT = N_PAD // NS
ZR = 128


def _sc_agg_body(h_hbm, row_hbm, col_hbm, acc_out,
                 rowbuf, colbuf, gbuf, zrow, acc_sh, sem):
    c = lax.axis_index("c")
    s = lax.axis_index("s")
    wid = s * NC + c
    base = wid * EPW

    def initz(i, carry):
        for j in range(D // 16):
            zrow[i, pl.ds(j * 16, 16)] = jnp.zeros((16,), jnp.float32)
        return carry
    lax.fori_loop(0, ZR, initz, 0)

    r0 = s * RPT
    for t in range(RPT // ZR):
        pltpu.sync_copy(zrow, acc_sh.at[pl.ds(r0 + t * ZR, ZR)])
    plsc.subcore_barrier()

    def step(i, carry):
        off = base + i * C
        pltpu.sync_copy(row_hbm.at[pl.ds(off, C)], rowbuf)
        pltpu.sync_copy(col_hbm.at[pl.ds(off, C)], colbuf)
        pltpu.async_copy(h_hbm.at[colbuf], gbuf, sem).wait()
        pltpu.sync_copy(gbuf, acc_sh.at[rowbuf], add=True)
        return carry
    lax.fori_loop(0, ITERS, step, 0)

    plsc.subcore_barrier()

    pltpu.sync_copy(acc_sh.at[pl.ds(r0, RPT)],
                    acc_out.at[pl.ds(c * N_PAD + r0, RPT)])


_sc_agg = pl.kernel(
    _sc_agg_body,
    out_type=jax.ShapeDtypeStruct((NC * N_PAD, D), jnp.float32),
    mesh=plsc.VectorSubcoreMesh(core_axis_name="c", subcore_axis_name="s"),
    scratch_types=[
        pltpu.VMEM((C,), jnp.int32),
        pltpu.VMEM((C,), jnp.int32),
        pltpu.VMEM((C, D), jnp.float32),
        pltpu.VMEM((ZR, D), jnp.float32),
        pltpu.VMEM_SHARED((N_PAD, D), jnp.float32),
        pltpu.SemaphoreType.DMA,
    ],
    name="sc_agg",
)


def _sc_deg_body(row_hbm, deg_out, rowbuf, ones_b, zdeg, deg_sh):
    c = lax.axis_index("c")
    s = lax.axis_index("s")
    wid = s * NC + c
    base = wid * EPW

    def initz(i, carry):
        for j in range(D // 16):
            zdeg[i, pl.ds(j * 16, 16)] = jnp.zeros((16,), jnp.float32)
        return carry
    lax.fori_loop(0, ZR, initz, 0)

    def initone(i, carry):
        for j in range(D // 16):
            ones_b[i, pl.ds(j * 16, 16)] = jnp.ones((16,), jnp.float32)
        return carry
    lax.fori_loop(0, C, initone, 0)

    r0 = s * RPT
    for t in range(RPT // ZR):
        pltpu.sync_copy(zdeg, deg_sh.at[pl.ds(r0 + t * ZR, ZR)])
    plsc.subcore_barrier()

    def step(i, carry):
        off = base + i * C
        pltpu.sync_copy(row_hbm.at[pl.ds(off, C)], rowbuf)
        pltpu.sync_copy(ones_b, deg_sh.at[rowbuf], add=True)
        return carry
    lax.fori_loop(0, ITERS, step, 0)

    plsc.subcore_barrier()

    pltpu.sync_copy(deg_sh.at[pl.ds(r0, RPT)],
                    deg_out.at[pl.ds(c * N_PAD + r0, RPT)])


_sc_deg = pl.kernel(
    _sc_deg_body,
    out_type=jax.ShapeDtypeStruct((NC * N_PAD, D), jnp.float32),
    mesh=plsc.VectorSubcoreMesh(core_axis_name="c", subcore_axis_name="s"),
    scratch_types=[
        pltpu.VMEM((C,), jnp.int32),
        pltpu.VMEM((C, D), jnp.float32),
        pltpu.VMEM((ZR, D), jnp.float32),
        pltpu.VMEM_SHARED((N_PAD, D), jnp.float32),
    ],
    name="sc_deg",
)


def _in_body(x_ref, w_ref, b_ref, o_ref):
    o_ref[...] = (jnp.dot(x_ref[...], w_ref[...],
                          preferred_element_type=jnp.float32) + b_ref[...])


def _layer_body(p0, p1, d0, d1, h_ref, w_ref, b_ref, g_ref, be_ref, o_ref):
    deg = jnp.maximum(d0[:, 0:1] + d1[:, 0:1], 1.0)
    agg = (p0[...] + p1[...]) / deg
    t = (h_ref[...] + jnp.dot(agg, w_ref[...],
                              preferred_element_type=jnp.float32) + b_ref[...])
    mu = jnp.mean(t, axis=1, keepdims=True)
    var = jnp.mean((t - mu) ** 2, axis=1, keepdims=True)
    y = (t - mu) * lax.rsqrt(var + 1e-5) * g_ref[...] + be_ref[...]
    o_ref[...] = jnp.maximum(y, 0.0)


def _layer_out_body(p0, p1, d0, d1, h_ref, w_ref, b_ref, g_ref, be_ref,
                    w2_ref, b2_ref, o_ref):
    deg = jnp.maximum(d0[:, 0:1] + d1[:, 0:1], 1.0)
    agg = (p0[...] + p1[...]) / deg
    t = (h_ref[...] + jnp.dot(agg, w_ref[...],
                              preferred_element_type=jnp.float32) + b_ref[...])
    mu = jnp.mean(t, axis=1, keepdims=True)
    var = jnp.mean((t - mu) ** 2, axis=1, keepdims=True)
    y = (t - mu) * lax.rsqrt(var + 1e-5) * g_ref[...] + be_ref[...]
    y = jnp.maximum(y, 0.0)
    o_ref[...] = (jnp.dot(y, w2_ref[...],
                          preferred_element_type=jnp.float32) + b2_ref[...])


_R = 1024


def _tc_in(x, W, b):
    return pl.pallas_call(
        _in_body,
        grid=(N_PAD // _R,),
        in_specs=[pl.BlockSpec((_R, D), lambda i: (i, 0)),
                  pl.BlockSpec((D, D), lambda i: (0, 0)),
                  pl.BlockSpec((1, D), lambda i: (0, 0))],
        out_specs=pl.BlockSpec((_R, D), lambda i: (i, 0)),
        out_shape=jax.ShapeDtypeStruct((N_PAD, D), jnp.float32),
    )(x, W, b.reshape(1, D))


def _part_specs():
    return [pl.BlockSpec((_R, D), lambda i: (i, 0)),
            pl.BlockSpec((_R, D), lambda i: (i + N_PAD // _R, 0)),
            pl.BlockSpec((_R, D), lambda i: (i, 0)),
            pl.BlockSpec((_R, D), lambda i: (i + N_PAD // _R, 0))]


def _tc_layer(acc, deg, h, W, b, g, be):
    specs = _part_specs() + [
        pl.BlockSpec((_R, D), lambda i: (i, 0)),
        pl.BlockSpec((D, D), lambda i: (0, 0)),
        pl.BlockSpec((1, D), lambda i: (0, 0)),
        pl.BlockSpec((1, D), lambda i: (0, 0)),
        pl.BlockSpec((1, D), lambda i: (0, 0)),
    ]
    return pl.pallas_call(
        _layer_body,
        grid=(N_PAD // _R,),
        in_specs=specs,
        out_specs=pl.BlockSpec((_R, D), lambda i: (i, 0)),
        out_shape=jax.ShapeDtypeStruct((N_PAD, D), jnp.float32),
    )(acc, acc, deg, deg, h, W, b.reshape(1, D), g.reshape(1, D),
      be.reshape(1, D))


def _tc_layer_out(acc, deg, h, W, b, g, be, W2, b2):
    specs = _part_specs() + [
        pl.BlockSpec((_R, D), lambda i: (i, 0)),
        pl.BlockSpec((D, D), lambda i: (0, 0)),
        pl.BlockSpec((1, D), lambda i: (0, 0)),
        pl.BlockSpec((1, D), lambda i: (0, 0)),
        pl.BlockSpec((1, D), lambda i: (0, 0)),
        pl.BlockSpec((D, D), lambda i: (0, 0)),
        pl.BlockSpec((1, D), lambda i: (0, 0)),
    ]
    return pl.pallas_call(
        _layer_out_body,
        grid=(N_PAD // _R,),
        in_specs=specs,
        out_specs=pl.BlockSpec((_R, D), lambda i: (i, 0)),
        out_shape=jax.ShapeDtypeStruct((N_PAD, D), jnp.float32),
    )(acc, acc, deg, deg, h, W, b.reshape(1, D), g.reshape(1, D),
      be.reshape(1, D), W2, b2.reshape(1, D))


def kernel(x, edge_index, W_in, b_in, W_c0, b_c0, W_c1, b_c1,
           g0, be0, g1, be1, W_out, b_out):
    row = edge_index[0]
    col = edge_index[1]
    x_p = jnp.pad(x, ((0, N_PAD - N), (0, 0)))
    deg = _sc_deg(row)
    h0 = _tc_in(x_p, W_in, b_in)
    acc0 = _sc_agg(h0, row, col)
    h1 = _tc_layer(acc0, deg, h0, W_c0, b_c0, g0, be0)
    acc1 = _sc_agg(h1, row, col)
    out = _tc_layer_out(acc1, deg, h1, W_c1, b_c1, g1, be1, W_out, b_out)
    return out[:N]


# speedup vs baseline: 5.5959x; 1.2794x over previous
import functools

import jax
import jax.numpy as jnp
from jax import lax
from jax.experimental import pallas as pl
from jax.experimental.pallas import tpu as pltpu
from jax.experimental.pallas import tpu_sc as plsc

N = 10000
E = 320000
D = 128

NC = 2
NS = 16
NW = NC * NS
EPW = E // NW
C = 80
ITERS = EPW // C
N_PAD = 10240
RPT = N_PAD // NS
ZR = 128


def _sc_agg_body(h_hbm, row_hbm, col_hbm, acc_out,
                 rowbuf0, colbuf0, gbuf0, rowbuf1, colbuf1, gbuf1,
                 zrow, acc_sh, sem0, sem1):
    c = lax.axis_index("c")
    s = lax.axis_index("s")
    wid = s * NC + c
    base = wid * EPW

    def initz(i, carry):
        for j in range(D // 16):
            zrow[i, pl.ds(j * 16, 16)] = jnp.zeros((16,), jnp.float32)
        return carry
    lax.fori_loop(0, ZR, initz, 0)

    r0 = s * RPT
    for t in range(RPT // ZR):
        pltpu.sync_copy(zrow, acc_sh.at[pl.ds(r0 + t * ZR, ZR)])
    plsc.subcore_barrier()

    def step(i, carry):
        off0 = base + (2 * i) * C
        off1 = off0 + C
        pltpu.sync_copy(row_hbm.at[pl.ds(off0, C)], rowbuf0)
        pltpu.sync_copy(col_hbm.at[pl.ds(off0, C)], colbuf0)
        d0 = pltpu.async_copy(h_hbm.at[colbuf0], gbuf0, sem0)
        pltpu.sync_copy(row_hbm.at[pl.ds(off1, C)], rowbuf1)
        pltpu.sync_copy(col_hbm.at[pl.ds(off1, C)], colbuf1)
        d1 = pltpu.async_copy(h_hbm.at[colbuf1], gbuf1, sem1)
        d0.wait()
        pltpu.sync_copy(gbuf0, acc_sh.at[rowbuf0], add=True)
        d1.wait()
        pltpu.sync_copy(gbuf1, acc_sh.at[rowbuf1], add=True)
        return carry
    lax.fori_loop(0, ITERS // 2, step, 0)

    offt = base + (ITERS - 1) * C
    pltpu.sync_copy(row_hbm.at[pl.ds(offt, C)], rowbuf0)
    pltpu.sync_copy(col_hbm.at[pl.ds(offt, C)], colbuf0)
    pltpu.async_copy(h_hbm.at[colbuf0], gbuf0, sem0).wait()
    pltpu.sync_copy(gbuf0, acc_sh.at[rowbuf0], add=True)

    plsc.subcore_barrier()

    pltpu.sync_copy(acc_sh.at[pl.ds(r0, RPT)],
                    acc_out.at[pl.ds(c * N_PAD + r0, RPT)])


_sc_agg = pl.kernel(
    _sc_agg_body,
    out_type=jax.ShapeDtypeStruct((NC * N_PAD, D), jnp.float32),
    mesh=plsc.VectorSubcoreMesh(core_axis_name="c", subcore_axis_name="s"),
    scratch_types=[
        pltpu.VMEM((C,), jnp.int32),
        pltpu.VMEM((C,), jnp.int32),
        pltpu.VMEM((C, D), jnp.float32),
        pltpu.VMEM((C,), jnp.int32),
        pltpu.VMEM((C,), jnp.int32),
        pltpu.VMEM((C, D), jnp.float32),
        pltpu.VMEM((ZR, D), jnp.float32),
        pltpu.VMEM_SHARED((N_PAD, D), jnp.float32),
        pltpu.SemaphoreType.DMA,
        pltpu.SemaphoreType.DMA,
    ],
    name="sc_agg",
)


def _sc_deg_body(row_hbm, deg_out, rowbuf, ones_b, zdeg, deg_sh):
    c = lax.axis_index("c")
    s = lax.axis_index("s")
    wid = s * NC + c
    base = wid * EPW

    def initz(i, carry):
        for j in range(D // 16):
            zdeg[i, pl.ds(j * 16, 16)] = jnp.zeros((16,), jnp.float32)
        return carry
    lax.fori_loop(0, ZR, initz, 0)

    def initone(i, carry):
        for j in range(D // 16):
            ones_b[i, pl.ds(j * 16, 16)] = jnp.ones((16,), jnp.float32)
        return carry
    lax.fori_loop(0, C, initone, 0)

    r0 = s * RPT
    for t in range(RPT // ZR):
        pltpu.sync_copy(zdeg, deg_sh.at[pl.ds(r0 + t * ZR, ZR)])
    plsc.subcore_barrier()

    def step(i, carry):
        off = base + i * C
        pltpu.sync_copy(row_hbm.at[pl.ds(off, C)], rowbuf)
        pltpu.sync_copy(ones_b, deg_sh.at[rowbuf], add=True)
        return carry
    lax.fori_loop(0, ITERS, step, 0)

    plsc.subcore_barrier()

    pltpu.sync_copy(deg_sh.at[pl.ds(r0, RPT)],
                    deg_out.at[pl.ds(c * N_PAD + r0, RPT)])


_sc_deg = pl.kernel(
    _sc_deg_body,
    out_type=jax.ShapeDtypeStruct((NC * N_PAD, D), jnp.float32),
    mesh=plsc.VectorSubcoreMesh(core_axis_name="c", subcore_axis_name="s"),
    scratch_types=[
        pltpu.VMEM((C,), jnp.int32),
        pltpu.VMEM((C, D), jnp.float32),
        pltpu.VMEM((ZR, D), jnp.float32),
        pltpu.VMEM_SHARED((N_PAD, D), jnp.float32),
    ],
    name="sc_deg",
)


def _in_body(x_ref, w_ref, b_ref, o_ref):
    o_ref[...] = (jnp.dot(x_ref[...], w_ref[...],
                          preferred_element_type=jnp.float32) + b_ref[...])


def _layer_body(p0, p1, d0, d1, h_ref, w_ref, b_ref, g_ref, be_ref, o_ref):
    deg = jnp.maximum(d0[:, 0:1] + d1[:, 0:1], 1.0)
    agg = (p0[...] + p1[...]) / deg
    t = (h_ref[...] + jnp.dot(agg, w_ref[...],
                              preferred_element_type=jnp.float32) + b_ref[...])
    mu = jnp.mean(t, axis=1, keepdims=True)
    var = jnp.mean((t - mu) ** 2, axis=1, keepdims=True)
    y = (t - mu) * lax.rsqrt(var + 1e-5) * g_ref[...] + be_ref[...]
    o_ref[...] = jnp.maximum(y, 0.0)


def _layer_out_body(p0, p1, d0, d1, h_ref, w_ref, b_ref, g_ref, be_ref,
                    w2_ref, b2_ref, o_ref):
    deg = jnp.maximum(d0[:, 0:1] + d1[:, 0:1], 1.0)
    agg = (p0[...] + p1[...]) / deg
    t = (h_ref[...] + jnp.dot(agg, w_ref[...],
                              preferred_element_type=jnp.float32) + b_ref[...])
    mu = jnp.mean(t, axis=1, keepdims=True)
    var = jnp.mean((t - mu) ** 2, axis=1, keepdims=True)
    y = (t - mu) * lax.rsqrt(var + 1e-5) * g_ref[...] + be_ref[...]
    y = jnp.maximum(y, 0.0)
    o_ref[...] = (jnp.dot(y, w2_ref[...],
                          preferred_element_type=jnp.float32) + b2_ref[...])


_R = 1024


def _tc_in(x, W, b):
    return pl.pallas_call(
        _in_body,
        grid=(N_PAD // _R,),
        in_specs=[pl.BlockSpec((_R, D), lambda i: (i, 0)),
                  pl.BlockSpec((D, D), lambda i: (0, 0)),
                  pl.BlockSpec((1, D), lambda i: (0, 0))],
        out_specs=pl.BlockSpec((_R, D), lambda i: (i, 0)),
        out_shape=jax.ShapeDtypeStruct((N_PAD, D), jnp.float32),
    )(x, W, b.reshape(1, D))


def _part_specs():
    return [pl.BlockSpec((_R, D), lambda i: (i, 0)),
            pl.BlockSpec((_R, D), lambda i: (i + N_PAD // _R, 0)),
            pl.BlockSpec((_R, D), lambda i: (i, 0)),
            pl.BlockSpec((_R, D), lambda i: (i + N_PAD // _R, 0))]


def _tc_layer(acc, deg, h, W, b, g, be):
    specs = _part_specs() + [
        pl.BlockSpec((_R, D), lambda i: (i, 0)),
        pl.BlockSpec((D, D), lambda i: (0, 0)),
        pl.BlockSpec((1, D), lambda i: (0, 0)),
        pl.BlockSpec((1, D), lambda i: (0, 0)),
        pl.BlockSpec((1, D), lambda i: (0, 0)),
    ]
    return pl.pallas_call(
        _layer_body,
        grid=(N_PAD // _R,),
        in_specs=specs,
        out_specs=pl.BlockSpec((_R, D), lambda i: (i, 0)),
        out_shape=jax.ShapeDtypeStruct((N_PAD, D), jnp.float32),
    )(acc, acc, deg, deg, h, W, b.reshape(1, D), g.reshape(1, D),
      be.reshape(1, D))


def _tc_layer_out(acc, deg, h, W, b, g, be, W2, b2):
    specs = _part_specs() + [
        pl.BlockSpec((_R, D), lambda i: (i, 0)),
        pl.BlockSpec((D, D), lambda i: (0, 0)),
        pl.BlockSpec((1, D), lambda i: (0, 0)),
        pl.BlockSpec((1, D), lambda i: (0, 0)),
        pl.BlockSpec((1, D), lambda i: (0, 0)),
        pl.BlockSpec((D, D), lambda i: (0, 0)),
        pl.BlockSpec((1, D), lambda i: (0, 0)),
    ]
    return pl.pallas_call(
        _layer_out_body,
        grid=(N_PAD // _R,),
        in_specs=specs,
        out_specs=pl.BlockSpec((_R, D), lambda i: (i, 0)),
        out_shape=jax.ShapeDtypeStruct((N_PAD, D), jnp.float32),
    )(acc, acc, deg, deg, h, W, b.reshape(1, D), g.reshape(1, D),
      be.reshape(1, D), W2, b2.reshape(1, D))


def kernel(x, edge_index, W_in, b_in, W_c0, b_c0, W_c1, b_c1,
           g0, be0, g1, be1, W_out, b_out):
    row = edge_index[0]
    col = edge_index[1]
    x_p = jnp.pad(x, ((0, N_PAD - N), (0, 0)))
    deg = _sc_deg(row)
    h0 = _tc_in(x_p, W_in, b_in)
    acc0 = _sc_agg(h0, row, col)
    h1 = _tc_layer(acc0, deg, h0, W_c0, b_c0, g0, be0)
    acc1 = _sc_agg(h1, row, col)
    out = _tc_layer_out(acc1, deg, h1, W_c1, b_c1, g1, be1, W_out, b_out)
    return out[:N]


# speedup vs baseline: 6.2708x; 1.1206x over previous
import functools

import jax
import jax.numpy as jnp
from jax import lax
from jax.experimental import pallas as pl
from jax.experimental.pallas import tpu as pltpu
from jax.experimental.pallas import tpu_sc as plsc

N = 10000
E = 320000
D = 128

NC = 2
NS = 16
NW = NC * NS
EPW = E // NW
C = 80
ITERS = EPW // C
N_PAD = 10240
RPT = N_PAD // NS
ZR = 16
NB = 4


def _sc_agg_body(h_hbm, row_hbm, col_hbm, acc_out,
                 rowbuf0, colbuf0, gbuf0, rowbuf1, colbuf1, gbuf1,
                 rowbuf2, colbuf2, gbuf2, rowbuf3, colbuf3, gbuf3,
                 zrow, acc_sh, sem0, sem1, sem2, sem3):
    rbs = (rowbuf0, rowbuf1, rowbuf2, rowbuf3)
    cbs = (colbuf0, colbuf1, colbuf2, colbuf3)
    gbs = (gbuf0, gbuf1, gbuf2, gbuf3)
    sems = (sem0, sem1, sem2, sem3)
    c = lax.axis_index("c")
    s = lax.axis_index("s")
    wid = s * NC + c
    base = wid * EPW

    def initz(i, carry):
        for j in range(D // 16):
            zrow[i, pl.ds(j * 16, 16)] = jnp.zeros((16,), jnp.float32)
        return carry
    lax.fori_loop(0, ZR, initz, 0)

    r0 = s * RPT
    for t in range(RPT // ZR):
        pltpu.sync_copy(zrow, acc_sh.at[pl.ds(r0 + t * ZR, ZR)])
    plsc.subcore_barrier()

    def step(i, carry):
        ds = []
        for b in range(NB):
            off = base + (NB * i + b) * C
            pltpu.sync_copy(row_hbm.at[pl.ds(off, C)], rbs[b])
            pltpu.sync_copy(col_hbm.at[pl.ds(off, C)], cbs[b])
            ds.append(pltpu.async_copy(h_hbm.at[cbs[b]], gbs[b], sems[b]))
        for b in range(NB):
            ds[b].wait()
            pltpu.sync_copy(gbs[b], acc_sh.at[rbs[b]], add=True)
        return carry
    lax.fori_loop(0, ITERS // NB, step, 0)

    for b in range(ITERS % NB):
        offt = base + ((ITERS // NB) * NB + b) * C
        pltpu.sync_copy(row_hbm.at[pl.ds(offt, C)], rbs[b])
        pltpu.sync_copy(col_hbm.at[pl.ds(offt, C)], cbs[b])
        pltpu.async_copy(h_hbm.at[cbs[b]], gbs[b], sems[b]).wait()
        pltpu.sync_copy(gbs[b], acc_sh.at[rbs[b]], add=True)

    plsc.subcore_barrier()

    pltpu.sync_copy(acc_sh.at[pl.ds(r0, RPT)],
                    acc_out.at[pl.ds(c * N_PAD + r0, RPT)])


_sc_agg = pl.kernel(
    _sc_agg_body,
    out_type=jax.ShapeDtypeStruct((NC * N_PAD, D), jnp.float32),
    mesh=plsc.VectorSubcoreMesh(core_axis_name="c", subcore_axis_name="s"),
    scratch_types=[
        pltpu.VMEM((C,), jnp.int32),
        pltpu.VMEM((C,), jnp.int32),
        pltpu.VMEM((C, D), jnp.float32),
    ] * NB + [
        pltpu.VMEM((ZR, D), jnp.float32),
        pltpu.VMEM_SHARED((N_PAD, D), jnp.float32),
    ] + [pltpu.SemaphoreType.DMA] * NB,
    name="sc_agg",
)


def _sc_deg_body(row_hbm, deg_out, rowbuf, ones_b, zdeg, deg_sh):
    c = lax.axis_index("c")
    s = lax.axis_index("s")
    wid = s * NC + c
    base = wid * EPW

    def initz(i, carry):
        for j in range(D // 16):
            zdeg[i, pl.ds(j * 16, 16)] = jnp.zeros((16,), jnp.float32)
        return carry
    lax.fori_loop(0, ZR, initz, 0)

    def initone(i, carry):
        for j in range(D // 16):
            ones_b[i, pl.ds(j * 16, 16)] = jnp.ones((16,), jnp.float32)
        return carry
    lax.fori_loop(0, C, initone, 0)

    r0 = s * RPT
    for t in range(RPT // ZR):
        pltpu.sync_copy(zdeg, deg_sh.at[pl.ds(r0 + t * ZR, ZR)])
    plsc.subcore_barrier()

    def step(i, carry):
        off = base + i * C
        pltpu.sync_copy(row_hbm.at[pl.ds(off, C)], rowbuf)
        pltpu.sync_copy(ones_b, deg_sh.at[rowbuf], add=True)
        return carry
    lax.fori_loop(0, ITERS, step, 0)

    plsc.subcore_barrier()

    pltpu.sync_copy(deg_sh.at[pl.ds(r0, RPT)],
                    deg_out.at[pl.ds(c * N_PAD + r0, RPT)])


_sc_deg = pl.kernel(
    _sc_deg_body,
    out_type=jax.ShapeDtypeStruct((NC * N_PAD, D), jnp.float32),
    mesh=plsc.VectorSubcoreMesh(core_axis_name="c", subcore_axis_name="s"),
    scratch_types=[
        pltpu.VMEM((C,), jnp.int32),
        pltpu.VMEM((C, D), jnp.float32),
        pltpu.VMEM((ZR, D), jnp.float32),
        pltpu.VMEM_SHARED((N_PAD, D), jnp.float32),
    ],
    name="sc_deg",
)


def _in_body(x_ref, w_ref, b_ref, o_ref):
    o_ref[...] = (jnp.dot(x_ref[...], w_ref[...],
                          preferred_element_type=jnp.float32) + b_ref[...])


def _layer_body(p0, p1, d0, d1, h_ref, w_ref, b_ref, g_ref, be_ref, o_ref):
    deg = jnp.maximum(d0[:, 0:1] + d1[:, 0:1], 1.0)
    agg = (p0[...] + p1[...]) / deg
    t = (h_ref[...] + jnp.dot(agg, w_ref[...],
                              preferred_element_type=jnp.float32) + b_ref[...])
    mu = jnp.mean(t, axis=1, keepdims=True)
    var = jnp.mean((t - mu) ** 2, axis=1, keepdims=True)
    y = (t - mu) * lax.rsqrt(var + 1e-5) * g_ref[...] + be_ref[...]
    o_ref[...] = jnp.maximum(y, 0.0)


def _layer_out_body(p0, p1, d0, d1, h_ref, w_ref, b_ref, g_ref, be_ref,
                    w2_ref, b2_ref, o_ref):
    deg = jnp.maximum(d0[:, 0:1] + d1[:, 0:1], 1.0)
    agg = (p0[...] + p1[...]) / deg
    t = (h_ref[...] + jnp.dot(agg, w_ref[...],
                              preferred_element_type=jnp.float32) + b_ref[...])
    mu = jnp.mean(t, axis=1, keepdims=True)
    var = jnp.mean((t - mu) ** 2, axis=1, keepdims=True)
    y = (t - mu) * lax.rsqrt(var + 1e-5) * g_ref[...] + be_ref[...]
    y = jnp.maximum(y, 0.0)
    o_ref[...] = (jnp.dot(y, w2_ref[...],
                          preferred_element_type=jnp.float32) + b2_ref[...])


_R = 1024


def _tc_in(x, W, b):
    return pl.pallas_call(
        _in_body,
        grid=(N_PAD // _R,),
        in_specs=[pl.BlockSpec((_R, D), lambda i: (i, 0)),
                  pl.BlockSpec((D, D), lambda i: (0, 0)),
                  pl.BlockSpec((1, D), lambda i: (0, 0))],
        out_specs=pl.BlockSpec((_R, D), lambda i: (i, 0)),
        out_shape=jax.ShapeDtypeStruct((N_PAD, D), jnp.float32),
    )(x, W, b.reshape(1, D))


def _part_specs():
    return [pl.BlockSpec((_R, D), lambda i: (i, 0)),
            pl.BlockSpec((_R, D), lambda i: (i + N_PAD // _R, 0)),
            pl.BlockSpec((_R, D), lambda i: (i, 0)),
            pl.BlockSpec((_R, D), lambda i: (i + N_PAD // _R, 0))]


def _tc_layer(acc, deg, h, W, b, g, be):
    specs = _part_specs() + [
        pl.BlockSpec((_R, D), lambda i: (i, 0)),
        pl.BlockSpec((D, D), lambda i: (0, 0)),
        pl.BlockSpec((1, D), lambda i: (0, 0)),
        pl.BlockSpec((1, D), lambda i: (0, 0)),
        pl.BlockSpec((1, D), lambda i: (0, 0)),
    ]
    return pl.pallas_call(
        _layer_body,
        grid=(N_PAD // _R,),
        in_specs=specs,
        out_specs=pl.BlockSpec((_R, D), lambda i: (i, 0)),
        out_shape=jax.ShapeDtypeStruct((N_PAD, D), jnp.float32),
    )(acc, acc, deg, deg, h, W, b.reshape(1, D), g.reshape(1, D),
      be.reshape(1, D))


def _tc_layer_out(acc, deg, h, W, b, g, be, W2, b2):
    specs = _part_specs() + [
        pl.BlockSpec((_R, D), lambda i: (i, 0)),
        pl.BlockSpec((D, D), lambda i: (0, 0)),
        pl.BlockSpec((1, D), lambda i: (0, 0)),
        pl.BlockSpec((1, D), lambda i: (0, 0)),
        pl.BlockSpec((1, D), lambda i: (0, 0)),
        pl.BlockSpec((D, D), lambda i: (0, 0)),
        pl.BlockSpec((1, D), lambda i: (0, 0)),
    ]
    return pl.pallas_call(
        _layer_out_body,
        grid=(N_PAD // _R,),
        in_specs=specs,
        out_specs=pl.BlockSpec((_R, D), lambda i: (i, 0)),
        out_shape=jax.ShapeDtypeStruct((N_PAD, D), jnp.float32),
    )(acc, acc, deg, deg, h, W, b.reshape(1, D), g.reshape(1, D),
      be.reshape(1, D), W2, b2.reshape(1, D))


def kernel(x, edge_index, W_in, b_in, W_c0, b_c0, W_c1, b_c1,
           g0, be0, g1, be1, W_out, b_out):
    row = edge_index[0]
    col = edge_index[1]
    x_p = jnp.pad(x, ((0, N_PAD - N), (0, 0)))
    deg = _sc_deg(row)
    h0 = _tc_in(x_p, W_in, b_in)
    acc0 = _sc_agg(h0, row, col)
    h1 = _tc_layer(acc0, deg, h0, W_c0, b_c0, g0, be0)
    acc1 = _sc_agg(h1, row, col)
    out = _tc_layer_out(acc1, deg, h1, W_c1, b_c1, g1, be1, W_out, b_out)
    return out[:N]
